# baseline (speedup 1.0000x reference)
import functools

import jax
import jax.numpy as jnp
from jax import lax
from jax.experimental import pallas as pl
from jax.experimental.pallas import tpu as pltpu
from jax.experimental.pallas import tpu_sc as plsc

_B, _L, _V = 1024, 50, 1000
_N = _B * _L
_VP = 1024

_info = plsc.get_sparse_core_info()
_NC, _NS, _LANES = _info.num_cores, _info.num_subcores, _info.num_lanes
_NW = _NC * _NS
_RW = _N // _NW
_GCH = 32
_GNCH = _RW // _GCH
_CH = 64
_NCH = _RW // _CH


def _lse_body(table_ref, out_ref):
    t = table_ref[...]
    m = jnp.max(t, axis=1)
    s = jnp.sum(jnp.exp(t - m[:, None]), axis=1)
    out_ref[...] = m + jnp.log(s)


def _row_lse(table):
    return pl.pallas_call(
        _lse_body,
        out_shape=jax.ShapeDtypeStruct((_V,), jnp.float32),
    )(table)


_SLICE_BLK = 2048


def _slice_body(in_ref, out_ref):
    out_ref[...] = in_ref[:, :_V]


def _strip_pad(logits_pad):
    return pl.pallas_call(
        _slice_body,
        grid=(_N // _SLICE_BLK,),
        in_specs=[pl.BlockSpec((_SLICE_BLK, _VP), lambda i: (i, 0))],
        out_specs=pl.BlockSpec((_SLICE_BLK, _V), lambda i: (i, 0)),
        out_shape=jax.ShapeDtypeStruct((_N, _V), jnp.float32),
    )(logits_pad)


def _flat_body(table_hbm, tflat_hbm, stage_v, sem):
    t = lax.axis_index("s") * _NC + lax.axis_index("c")
    start = 31 * t + jnp.minimum(t, 8)
    count = jnp.where(t < 8, 32, 31)

    def fire_in(j, carry):
        @pl.when(j < count)
        def _():
            pltpu.async_copy(table_hbm.at[start + j], stage_v.at[j], sem)
        return carry

    def drain_in(j, carry):
        @pl.when(j < count)
        def _():
            pltpu.make_async_copy(table_hbm.at[0], stage_v.at[0], sem).wait()
        return carry

    def fire_out(j, carry):
        @pl.when(j < count)
        def _():
            pltpu.async_copy(stage_v.at[j],
                             tflat_hbm.at[pl.ds((start + j) * _V, _V)], sem)
        return carry

    def drain_out(j, carry):
        @pl.when(j < count)
        def _():
            pltpu.make_async_copy(stage_v.at[0],
                                  tflat_hbm.at[pl.ds(0, _V)], sem).wait()
        return carry

    lax.fori_loop(0, 32, fire_in, 0)
    lax.fori_loop(0, 32, drain_in, 0)
    lax.fori_loop(0, 32, fire_out, 0)
    lax.fori_loop(0, 32, drain_out, 0)


def _flatten_table(table):
    mesh = plsc.VectorSubcoreMesh(core_axis_name="c", subcore_axis_name="s")
    fn = pl.kernel(
        _flat_body,
        out_type=jax.ShapeDtypeStruct((_V * _V,), jnp.float32),
        mesh=mesh,
        compiler_params=pltpu.CompilerParams(use_tc_tiling_on_sc=False),
        scratch_types=[
            pltpu.VMEM((32, _V), jnp.float32),
            pltpu.SemaphoreType.DMA,
        ],
    )
    return fn(table)


def _gather_body(idx_hbm, table_hbm, out_hbm, idx_v, rows0, rows1,
                 gsem0, gsem1, osem0, osem1):
    wid = lax.axis_index("s") * _NC + lax.axis_index("c")
    base = wid * _RW
    pltpu.sync_copy(idx_hbm.at[wid], idx_v)

    def fire_gather(c, buf, sem):
        for j in range(_GCH // _LANES):
            iv = idx_v[pl.ds(c * _GCH + j * _LANES, _LANES)]
            pltpu.async_copy(table_hbm.at[iv],
                             buf.at[pl.ds(j * _LANES, _LANES)], sem)

    def wait_gather(buf, sem):
        for j in range(_GCH // _LANES):
            pltpu.make_async_copy(table_hbm.at[idx_v[pl.ds(0, _LANES)]],
                                  buf.at[pl.ds(0, _LANES)], sem).wait()

    def fire_out(c, buf, sem):
        pltpu.async_copy(buf, out_hbm.at[pl.ds(base + c * _GCH, _GCH)], sem)

    def wait_out(buf, sem):
        pltpu.make_async_copy(buf, out_hbm.at[pl.ds(base, _GCH)], sem).wait()

    fire_gather(0, rows0, gsem0)
    fire_gather(1, rows1, gsem1)

    def pair(i, carry):
        c0 = 2 * i
        wait_gather(rows0, gsem0)
        fire_out(c0, rows0, osem0)
        wait_gather(rows1, gsem1)
        fire_out(c0 + 1, rows1, osem1)
        wait_out(rows0, osem0)
        wait_out(rows1, osem1)

        @pl.when(i < _GNCH // 2 - 1)
        def _():
            fire_gather(c0 + 2, rows0, gsem0)
            fire_gather(c0 + 3, rows1, gsem1)

        return carry

    lax.fori_loop(0, _GNCH // 2, pair, 0)


@jax.jit
def _gather_call(idx2, table_pad):
    mesh = plsc.VectorSubcoreMesh(core_axis_name="c", subcore_axis_name="s")
    fn = pl.kernel(
        _gather_body,
        out_type=jax.ShapeDtypeStruct((_N, _VP), jnp.float32),
        mesh=mesh,
        scratch_types=[
            pltpu.VMEM((_RW,), jnp.int32),
            pltpu.VMEM((_GCH, _VP), jnp.float32),
            pltpu.VMEM((_GCH, _VP), jnp.float32),
            pltpu.SemaphoreType.DMA,
            pltpu.SemaphoreType.DMA,
            pltpu.SemaphoreType.DMA,
            pltpu.SemaphoreType.DMA,
        ],
    )
    return fn(idx2, table_pad)


_LCH = 128
_LNCH = _RW // _LCH
_LT = _RW - _LNCH * _LCH


def _loss_body(idx_hbm, tgt_hbm, tflat_hbm, lse_hbm, part_hbm,
               idx_v, tgt_v, comb_v, lse_v, tgtv_v, part_v, lsem):
    wid = lax.axis_index("s") * _NC + lax.axis_index("c")
    pltpu.sync_copy(idx_hbm.at[wid], idx_v)
    pltpu.sync_copy(tgt_hbm.at[wid], tgt_v)

    def comb(k, carry):
        sl = pl.ds(k * _LANES, _LANES)
        comb_v[sl] = idx_v[sl] * _V + tgt_v[sl]
        return carry

    lax.fori_loop(0, _RW // _LANES, comb, 0)

    for c in range(_LNCH):
        sl = pl.ds(c * _LCH, _LCH)
        pltpu.async_copy(lse_hbm.at[idx_v.at[sl]], lse_v.at[sl], lsem)
        pltpu.async_copy(tflat_hbm.at[comb_v.at[sl]], tgtv_v.at[sl], lsem)
    tl = pl.ds(_LNCH * _LCH, _LT)
    pltpu.async_copy(lse_hbm.at[idx_v.at[tl]], lse_v.at[tl], lsem)
    pltpu.async_copy(tflat_hbm.at[comb_v.at[tl]], tgtv_v.at[tl], lsem)
    for c in range(_LNCH):
        sl = pl.ds(c * _LCH, _LCH)
        pltpu.make_async_copy(lse_hbm.at[idx_v.at[sl]], lse_v.at[sl],
                              lsem).wait()
        pltpu.make_async_copy(tflat_hbm.at[comb_v.at[sl]], tgtv_v.at[sl],
                              lsem).wait()
    pltpu.make_async_copy(lse_hbm.at[idx_v.at[tl]], lse_v.at[tl], lsem).wait()
    pltpu.make_async_copy(tflat_hbm.at[comb_v.at[tl]], tgtv_v.at[tl],
                          lsem).wait()

    def red(k, acc):
        sl = pl.ds(k * _LANES, _LANES)
        return acc + (lse_v[sl] - tgtv_v[sl])

    part_v[...] = lax.fori_loop(0, _RW // _LANES, red,
                                jnp.zeros((_LANES,), jnp.float32))
    pltpu.sync_copy(part_v, part_hbm.at[wid])


@jax.jit
def _loss_call(idx2, tgt2, tflat, lse_row):
    mesh = plsc.VectorSubcoreMesh(core_axis_name="c", subcore_axis_name="s")
    fn = pl.kernel(
        _loss_body,
        out_type=jax.ShapeDtypeStruct((_NW, _LANES), jnp.float32),
        mesh=mesh,
        compiler_params=pltpu.CompilerParams(use_tc_tiling_on_sc=False),
        scratch_types=[
            pltpu.VMEM((_RW,), jnp.int32),
            pltpu.VMEM((_RW,), jnp.int32),
            pltpu.VMEM((_RW,), jnp.int32),
            pltpu.VMEM((_RW,), jnp.float32),
            pltpu.VMEM((_RW,), jnp.float32),
            pltpu.VMEM((_LANES,), jnp.float32),
            pltpu.SemaphoreType.DMA,
        ],
    )
    return fn(idx2, tgt2, tflat, lse_row)


def kernel(input_b_l, target_b_1, embedding_table):
    idx2 = input_b_l.astype(jnp.int32).reshape(_NW, _RW)
    tgt2 = target_b_1.astype(jnp.int32).reshape(_NW, _RW)
    tflat = _flatten_table(embedding_table)
    lse_row = _row_lse(embedding_table)
    table_pad = jnp.pad(embedding_table, ((0, 0), (0, _VP - _V)))
    logits_pad = _gather_call(idx2, table_pad)
    parts = _loss_call(idx2, tgt2, tflat, lse_row)
    loss = jnp.sum(parts) / _N
    return _strip_pad(logits_pad), loss


# speedup vs baseline: 1.4395x; 1.4395x over previous
import functools

import jax
import jax.numpy as jnp
from jax import lax
from jax.experimental import pallas as pl
from jax.experimental.pallas import tpu as pltpu
from jax.experimental.pallas import tpu_sc as plsc

_B, _L, _V = 1024, 50, 1000
_N = _B * _L
_VP = 1024

_info = plsc.get_sparse_core_info()
_NC, _NS, _LANES = _info.num_cores, _info.num_subcores, _info.num_lanes
_NW = _NC * _NS
_RW = _N // _NW
_GCH = 16
_GNCH = _RW // _GCH
_CH = 64
_NCH = _RW // _CH


def _lse_body(table_ref, out_ref):
    t = table_ref[...]
    m = jnp.max(t, axis=1)
    s = jnp.sum(jnp.exp(t - m[:, None]), axis=1)
    out_ref[...] = m + jnp.log(s)


def _row_lse(table):
    return pl.pallas_call(
        _lse_body,
        out_shape=jax.ShapeDtypeStruct((_V,), jnp.float32),
    )(table)


def _flat_body(table_hbm, tflat_hbm, stage_v, sem):
    t = lax.axis_index("s") * _NC + lax.axis_index("c")
    start = 31 * t + jnp.minimum(t, 8)
    count = jnp.where(t < 8, 32, 31)

    def fire_in(j, carry):
        @pl.when(j < count)
        def _():
            pltpu.async_copy(table_hbm.at[start + j], stage_v.at[j], sem)
        return carry

    def drain_in(j, carry):
        @pl.when(j < count)
        def _():
            pltpu.make_async_copy(table_hbm.at[0], stage_v.at[0], sem).wait()
        return carry

    def fire_out(j, carry):
        @pl.when(j < count)
        def _():
            pltpu.async_copy(stage_v.at[j],
                             tflat_hbm.at[pl.ds((start + j) * _V, _V)], sem)
        return carry

    def drain_out(j, carry):
        @pl.when(j < count)
        def _():
            pltpu.make_async_copy(stage_v.at[0],
                                  tflat_hbm.at[pl.ds(0, _V)], sem).wait()
        return carry

    lax.fori_loop(0, 32, fire_in, 0)
    lax.fori_loop(0, 32, drain_in, 0)
    lax.fori_loop(0, 32, fire_out, 0)
    lax.fori_loop(0, 32, drain_out, 0)


def _flatten_table(table):
    mesh = plsc.VectorSubcoreMesh(core_axis_name="c", subcore_axis_name="s")
    fn = pl.kernel(
        _flat_body,
        out_type=jax.ShapeDtypeStruct((_V * _V,), jnp.float32),
        mesh=mesh,
        compiler_params=pltpu.CompilerParams(use_tc_tiling_on_sc=False),
        scratch_types=[
            pltpu.VMEM((32, _V), jnp.float32),
            pltpu.SemaphoreType.DMA,
        ],
    )
    return fn(table)


_NBUF = 4


def _gather_body(idx_hbm, table_hbm, out_hbm, idx_v, rows, gsems, osems):
    wid = lax.axis_index("s") * _NC + lax.axis_index("c")
    base = wid * _RW
    pltpu.sync_copy(idx_hbm.at[wid], idx_v)

    def fire_gather(c, k):
        iv = idx_v[pl.ds(c * _GCH, _GCH)]
        pltpu.async_copy(table_hbm.at[iv], rows.at[k], gsems.at[k])

    def wait_gather(k):
        pltpu.make_async_copy(table_hbm.at[idx_v[pl.ds(0, _GCH)]],
                              rows.at[k], gsems.at[k]).wait()

    def fire_out(c, k):
        pltpu.async_copy(rows.at[k], out_hbm.at[pl.ds(base + c * _GCH, _GCH)],
                         osems.at[k])

    def wait_out(k):
        pltpu.make_async_copy(rows.at[k], out_hbm.at[pl.ds(base, _GCH)],
                              osems.at[k]).wait()

    for k in range(_NBUF):
        fire_gather(k, k)

    def step(i, carry):
        c0 = _NBUF * i
        for k in range(_NBUF):
            wait_gather(k)
            fire_out(c0 + k, k)

        @pl.when(i < _GNCH // _NBUF - 1)
        def _():
            for k in range(_NBUF):
                wait_out(k)
                fire_gather(c0 + _NBUF + k, k)

        return carry

    lax.fori_loop(0, _GNCH // _NBUF, step, 0)
    for k in range(_NBUF):
        wait_out(k)


@jax.jit
def _gather_call(idx2, table_pad):
    mesh = plsc.VectorSubcoreMesh(core_axis_name="c", subcore_axis_name="s")
    fn = pl.kernel(
        _gather_body,
        out_type=jax.ShapeDtypeStruct((_N, _VP), jnp.float32),
        mesh=mesh,
        scratch_types=[
            pltpu.VMEM((_RW,), jnp.int32),
            pltpu.VMEM((_NBUF, _GCH, _VP), jnp.float32),
            pltpu.SemaphoreType.DMA((_NBUF,)),
            pltpu.SemaphoreType.DMA((_NBUF,)),
        ],
    )
    return fn(idx2, table_pad)


def _loss_body(idx_hbm, tgt_hbm, tflat_hbm, lse_hbm, part_hbm,
               idx_v, tgt_c, comb_c, lse_c, tgtv_c, part_v, lsem, tsem):
    wid = lax.axis_index("s") * _NC + lax.axis_index("c")
    pltpu.sync_copy(idx_hbm.at[wid], idx_v)
    part_v[...] = jnp.zeros((_LANES,), jnp.float32)

    def chunk(c, carry):
        pltpu.sync_copy(tgt_hbm.at[wid, c], tgt_c)
        for j in range(_CH // _LANES):
            sl = pl.ds(j * _LANES, _LANES)
            comb_c[sl] = idx_v[c, sl] * _V + tgt_c[sl]
        pltpu.async_copy(lse_hbm.at[idx_v.at[c]], lse_c, lsem)
        pltpu.async_copy(tflat_hbm.at[comb_c], tgtv_c, tsem)
        pltpu.make_async_copy(lse_hbm.at[idx_v.at[c]], lse_c, lsem).wait()
        pltpu.make_async_copy(tflat_hbm.at[comb_c], tgtv_c, tsem).wait()
        acc = part_v[...]
        for j in range(_CH // _LANES):
            sl = pl.ds(j * _LANES, _LANES)
            acc = acc + (lse_c[sl] - tgtv_c[sl])
        part_v[...] = acc
        return carry

    lax.fori_loop(0, _NCH, chunk, 0)
    pltpu.sync_copy(part_v, part_hbm.at[wid])


@jax.jit
def _loss_call(idx3, tgt3, tflat, lse_row):
    mesh = plsc.VectorSubcoreMesh(core_axis_name="c", subcore_axis_name="s")
    fn = pl.kernel(
        _loss_body,
        out_type=jax.ShapeDtypeStruct((_NW, _LANES), jnp.float32),
        mesh=mesh,
        compiler_params=pltpu.CompilerParams(use_tc_tiling_on_sc=False),
        scratch_types=[
            pltpu.VMEM((_NCH, _CH), jnp.int32),
            pltpu.VMEM((_CH,), jnp.int32),
            pltpu.VMEM((_CH,), jnp.int32),
            pltpu.VMEM((_CH,), jnp.float32),
            pltpu.VMEM((_CH,), jnp.float32),
            pltpu.VMEM((_LANES,), jnp.float32),
            pltpu.SemaphoreType.DMA,
            pltpu.SemaphoreType.DMA,
        ],
    )
    return fn(idx3, tgt3, tflat, lse_row)


def kernel(input_b_l, target_b_1, embedding_table):
    idx2 = input_b_l.astype(jnp.int32).reshape(_NW, _RW)
    idx3 = input_b_l.astype(jnp.int32).reshape(_NW, _NCH, _CH)
    tgt3 = target_b_1.astype(jnp.int32).reshape(_NW, _NCH, _CH)
    tflat = _flatten_table(embedding_table)
    lse_row = _row_lse(embedding_table)
    table_pad = jnp.pad(embedding_table, ((0, 0), (0, _VP - _V)))
    logits_pad = _gather_call(idx2, table_pad)
    parts = _loss_call(idx3, tgt3, tflat, lse_row)
    loss = jnp.sum(parts) / _N
    return logits_pad[:, :_V], loss


# speedup vs baseline: 1.4459x; 1.0044x over previous
import functools

import jax
import jax.numpy as jnp
from jax import lax
from jax.experimental import pallas as pl
from jax.experimental.pallas import tpu as pltpu
from jax.experimental.pallas import tpu_sc as plsc

_B, _L, _V = 1024, 50, 1000
_N = _B * _L
_VP = 1024

_info = plsc.get_sparse_core_info()
_NC, _NS, _LANES = _info.num_cores, _info.num_subcores, _info.num_lanes
_NW = _NC * _NS
_RW = _N // _NW
_GCH = 16
_GNCH = _RW // _GCH
_CH = 64
_NCH = _RW // _CH


def _lse_body(table_ref, out_ref):
    t = table_ref[...]
    m = jnp.max(t, axis=1)
    s = jnp.sum(jnp.exp(t - m[:, None]), axis=1)
    out_ref[...] = m + jnp.log(s)


def _row_lse(table):
    return pl.pallas_call(
        _lse_body,
        out_shape=jax.ShapeDtypeStruct((_V,), jnp.float32),
    )(table)


def _flat_body(table_hbm, tflat_hbm, stage_v, sem):
    t = lax.axis_index("s") * _NC + lax.axis_index("c")
    start = 31 * t + jnp.minimum(t, 8)
    count = jnp.where(t < 8, 32, 31)

    def fire_in(j, carry):
        @pl.when(j < count)
        def _():
            pltpu.async_copy(table_hbm.at[start + j], stage_v.at[j], sem)
        return carry

    def drain_in(j, carry):
        @pl.when(j < count)
        def _():
            pltpu.make_async_copy(table_hbm.at[0], stage_v.at[0], sem).wait()
        return carry

    def fire_out(j, carry):
        @pl.when(j < count)
        def _():
            pltpu.async_copy(stage_v.at[j],
                             tflat_hbm.at[pl.ds((start + j) * _V, _V)], sem)
        return carry

    def drain_out(j, carry):
        @pl.when(j < count)
        def _():
            pltpu.make_async_copy(stage_v.at[0],
                                  tflat_hbm.at[pl.ds(0, _V)], sem).wait()
        return carry

    lax.fori_loop(0, 32, fire_in, 0)
    lax.fori_loop(0, 32, drain_in, 0)
    lax.fori_loop(0, 32, fire_out, 0)
    lax.fori_loop(0, 32, drain_out, 0)


def _flatten_table(table):
    mesh = plsc.VectorSubcoreMesh(core_axis_name="c", subcore_axis_name="s")
    fn = pl.kernel(
        _flat_body,
        out_type=jax.ShapeDtypeStruct((_V * _V,), jnp.float32),
        mesh=mesh,
        compiler_params=pltpu.CompilerParams(use_tc_tiling_on_sc=False),
        scratch_types=[
            pltpu.VMEM((32, _V), jnp.float32),
            pltpu.SemaphoreType.DMA,
        ],
    )
    return fn(table)


_NBUF = 5


def _gather_body(idx_hbm, table_hbm, out_hbm, idx_v, rows, gsems, osems):
    wid = lax.axis_index("s") * _NC + lax.axis_index("c")
    base = wid * _RW
    pltpu.sync_copy(idx_hbm.at[wid], idx_v)

    def fire_gather(c, k):
        iv = idx_v[pl.ds(c * _GCH, _GCH)]
        pltpu.async_copy(table_hbm.at[iv], rows.at[k], gsems.at[k])

    def wait_gather(k):
        pltpu.make_async_copy(table_hbm.at[idx_v[pl.ds(0, _GCH)]],
                              rows.at[k], gsems.at[k]).wait()

    def fire_out(c, k):
        pltpu.async_copy(rows.at[k], out_hbm.at[pl.ds(base + c * _GCH, _GCH)],
                         osems.at[k])

    def wait_out(k):
        pltpu.make_async_copy(rows.at[k], out_hbm.at[pl.ds(base, _GCH)],
                              osems.at[k]).wait()

    for k in range(_NBUF):
        fire_gather(k, k)

    def step(i, carry):
        c0 = _NBUF * i
        for k in range(_NBUF):
            wait_gather(k)
            fire_out(c0 + k, k)

        @pl.when(i < _GNCH // _NBUF - 1)
        def _():
            for k in range(_NBUF):
                wait_out(k)
                fire_gather(c0 + _NBUF + k, k)

        return carry

    lax.fori_loop(0, _GNCH // _NBUF, step, 0)
    for k in range(_NBUF):
        wait_out(k)


@jax.jit
def _gather_call(idx2, table_pad):
    mesh = plsc.VectorSubcoreMesh(core_axis_name="c", subcore_axis_name="s")
    fn = pl.kernel(
        _gather_body,
        out_type=jax.ShapeDtypeStruct((_N, _VP), jnp.float32),
        mesh=mesh,
        scratch_types=[
            pltpu.VMEM((_RW,), jnp.int32),
            pltpu.VMEM((_NBUF, _GCH, _VP), jnp.float32),
            pltpu.SemaphoreType.DMA((_NBUF,)),
            pltpu.SemaphoreType.DMA((_NBUF,)),
        ],
    )
    return fn(idx2, table_pad)


def _loss_body(idx_hbm, tgt_hbm, tflat_hbm, lse_hbm, part_hbm,
               idx_v, tgt_c, comb_c, lse_c, tgtv_c, part_v, lsem, tsem):
    wid = lax.axis_index("s") * _NC + lax.axis_index("c")
    pltpu.sync_copy(idx_hbm.at[wid], idx_v)
    part_v[...] = jnp.zeros((_LANES,), jnp.float32)

    def chunk(c, carry):
        pltpu.sync_copy(tgt_hbm.at[wid, c], tgt_c)
        for j in range(_CH // _LANES):
            sl = pl.ds(j * _LANES, _LANES)
            comb_c[sl] = idx_v[c, sl] * _V + tgt_c[sl]
        pltpu.async_copy(lse_hbm.at[idx_v.at[c]], lse_c, lsem)
        pltpu.async_copy(tflat_hbm.at[comb_c], tgtv_c, tsem)
        pltpu.make_async_copy(lse_hbm.at[idx_v.at[c]], lse_c, lsem).wait()
        pltpu.make_async_copy(tflat_hbm.at[comb_c], tgtv_c, tsem).wait()
        acc = part_v[...]
        for j in range(_CH // _LANES):
            sl = pl.ds(j * _LANES, _LANES)
            acc = acc + (lse_c[sl] - tgtv_c[sl])
        part_v[...] = acc
        return carry

    lax.fori_loop(0, _NCH, chunk, 0)
    pltpu.sync_copy(part_v, part_hbm.at[wid])


@jax.jit
def _loss_call(idx3, tgt3, tflat, lse_row):
    mesh = plsc.VectorSubcoreMesh(core_axis_name="c", subcore_axis_name="s")
    fn = pl.kernel(
        _loss_body,
        out_type=jax.ShapeDtypeStruct((_NW, _LANES), jnp.float32),
        mesh=mesh,
        compiler_params=pltpu.CompilerParams(use_tc_tiling_on_sc=False),
        scratch_types=[
            pltpu.VMEM((_NCH, _CH), jnp.int32),
            pltpu.VMEM((_CH,), jnp.int32),
            pltpu.VMEM((_CH,), jnp.int32),
            pltpu.VMEM((_CH,), jnp.float32),
            pltpu.VMEM((_CH,), jnp.float32),
            pltpu.VMEM((_LANES,), jnp.float32),
            pltpu.SemaphoreType.DMA,
            pltpu.SemaphoreType.DMA,
        ],
    )
    return fn(idx3, tgt3, tflat, lse_row)


def kernel(input_b_l, target_b_1, embedding_table):
    idx2 = input_b_l.astype(jnp.int32).reshape(_NW, _RW)
    idx3 = input_b_l.astype(jnp.int32).reshape(_NW, _NCH, _CH)
    tgt3 = target_b_1.astype(jnp.int32).reshape(_NW, _NCH, _CH)
    tflat = _flatten_table(embedding_table)
    lse_row = _row_lse(embedding_table)
    table_pad = jnp.pad(embedding_table, ((0, 0), (0, _VP - _V)))
    logits_pad = _gather_call(idx2, table_pad)
    parts = _loss_call(idx3, tgt3, tflat, lse_row)
    loss = jnp.sum(parts) / _N
    return logits_pad[:, :_V], loss
